# Initial kernel scaffold; baseline (speedup 1.0000x reference)
import jax
import jax.numpy as jnp
from jax.experimental import pallas as pl


def kernel(x, embedding_u):
    raise NotImplementedError("write your pallas kernel here")


# speedup vs baseline: 1.8558x; 1.8558x over previous
import functools

import jax
import jax.numpy as jnp
from jax import lax
from jax.experimental import pallas as pl
from jax.experimental.pallas import tpu as pltpu
from jax.experimental.pallas import tpu_sc as plsc

_NUM_ITEMS = 1000000
_D = 64
_BATCH = 16384
_HIST = 50
_TOT = _BATCH * _HIST

_NC = 2
_NS = 16
_NW = _NC * _NS
_BPW = _TOT // _NW
_C = 512
_NCH = _BPW // _C


def _gather_body(idx_hbm, table_hbm, out_hbm, idx0, idx1, rows0, rows1,
                 sem0, sem1):
    wid = lax.axis_index("s") * _NC + lax.axis_index("c")

    sems = (sem0, sem1)
    idxb = (idx0, idx1)
    rows = (rows0, rows1)

    def start(j, b):
        pltpu.sync_copy(idx_hbm.at[wid, j], idxb[b])
        pltpu.async_copy(table_hbm.at[idxb[b]], rows[b], sems[b])

    def finish(j, b):
        pltpu.make_async_copy(
            table_hbm.at[idxb[b]], rows[b], sems[b]
        ).wait()
        pltpu.sync_copy(rows[b], out_hbm.at[wid, j])

    start(0, 0)

    @pl.loop(0, _NCH, step=2)
    def _(j):
        start(j + 1, 1)
        finish(j, 0)

        @pl.when(j + 2 < _NCH)
        def _():
            start(j + 2, 0)

        finish(j + 1, 1)


@jax.jit
def _embedding_lookup(x_flat, table):
    mesh = plsc.VectorSubcoreMesh(core_axis_name="c", subcore_axis_name="s")
    call = functools.partial(
        pl.kernel,
        mesh=mesh,
        out_type=jax.ShapeDtypeStruct((_NW, _NCH, _C, _D), jnp.float32),
        scratch_types=[
            pltpu.VMEM((_C,), jnp.int32),
            pltpu.VMEM((_C,), jnp.int32),
            pltpu.VMEM((_C, _D), jnp.float32),
            pltpu.VMEM((_C, _D), jnp.float32),
            pltpu.SemaphoreType.DMA,
            pltpu.SemaphoreType.DMA,
        ],
        compiler_params=pltpu.CompilerParams(use_tc_tiling_on_sc=False),
    )(_gather_body)
    return call(x_flat, table)


def kernel(x, embedding_u):
    x_flat = x.reshape(_NW, _NCH, _C).astype(jnp.int32)
    out = _embedding_lookup(x_flat, embedding_u)
    return out.reshape(_BATCH, _HIST, _D)


# speedup vs baseline: 1.8772x; 1.0115x over previous
import functools

import jax
import jax.numpy as jnp
from jax import lax
from jax.experimental import pallas as pl
from jax.experimental.pallas import tpu as pltpu
from jax.experimental.pallas import tpu_sc as plsc

_NUM_ITEMS = 1000000
_D = 64
_BATCH = 16384
_HIST = 50
_TOT = _BATCH * _HIST

_NC = 2
_NS = 16
_NW = _NC * _NS
_BPW = _TOT // _NW
_C = 400
_NCH = _BPW // _C

_NBUF = 4


def _gather_body(idx_hbm, table_hbm, out_hbm, *scratch):
    idxb = scratch[0:_NBUF]
    rows = scratch[_NBUF:2 * _NBUF]
    isem = scratch[2 * _NBUF:3 * _NBUF]
    gsem = scratch[3 * _NBUF:4 * _NBUF]
    osem = scratch[4 * _NBUF:5 * _NBUF]

    wid = lax.axis_index("s") * _NC + lax.axis_index("c")

    def fetch_idx(j, s):
        pltpu.async_copy(idx_hbm.at[wid, j], idxb[s], isem[s])

    def wait_idx(j, s):
        pltpu.make_async_copy(idx_hbm.at[wid, j], idxb[s], isem[s]).wait()

    def start_gather(s):
        pltpu.async_copy(table_hbm.at[idxb[s]], rows[s], gsem[s])

    def wait_gather(s):
        pltpu.make_async_copy(table_hbm.at[idxb[s]], rows[s], gsem[s]).wait()

    def start_store(j, s):
        pltpu.async_copy(rows[s], out_hbm.at[wid, j], osem[s])

    def wait_store(j, s):
        pltpu.make_async_copy(rows[s], out_hbm.at[wid, j], osem[s]).wait()

    for s in range(_NBUF):
        fetch_idx(s, s)
    for s in range(2):
        wait_idx(s, s)
        start_gather(s)

    @pl.loop(0, _NCH, step=_NBUF)
    def _(j):
        for s in range(_NBUF):
            cur = j + s
            wait_gather(s)

            @pl.when(cur + _NBUF < _NCH)
            def _():
                fetch_idx(cur + _NBUF, s)

            start_store(cur, s)

            ns = (s + 2) % _NBUF
            nxt = cur + 2

            @pl.when(nxt < _NCH)
            def _():
                @pl.when(cur >= 2)
                def _():
                    wait_store(cur - 2, ns)

                wait_idx(nxt, ns)
                start_gather(ns)

    wait_store(_NCH - 2, (_NCH - 2) % _NBUF)
    wait_store(_NCH - 1, (_NCH - 1) % _NBUF)


@jax.jit
def _embedding_lookup(x_flat, table):
    mesh = plsc.VectorSubcoreMesh(core_axis_name="c", subcore_axis_name="s")
    call = functools.partial(
        pl.kernel,
        mesh=mesh,
        out_type=jax.ShapeDtypeStruct((_NW, _NCH, _C, _D), jnp.float32),
        scratch_types=(
            [pltpu.VMEM((_C,), jnp.int32) for _ in range(_NBUF)]
            + [pltpu.VMEM((_C, _D), jnp.float32) for _ in range(_NBUF)]
            + [pltpu.SemaphoreType.DMA for _ in range(3 * _NBUF)]
        ),
        compiler_params=pltpu.CompilerParams(use_tc_tiling_on_sc=False),
    )(_gather_body)
    return call(x_flat, table)


def kernel(x, embedding_u):
    x_flat = x.reshape(_NW, _NCH, _C).astype(jnp.int32)
    out = _embedding_lookup(x_flat, embedding_u)
    return out.reshape(_BATCH, _HIST, _D)
